# baseline (speedup 1.0000x reference)
import jax
import jax.numpy as jnp
from jax.experimental import pallas as pl
from jax.experimental.pallas import tpu as pltpu


def _body(x_ref, pos_ref, o_ref):
    R = x_ref.shape[0]
    p10 = pos_ref[0:10, :]
    pos40 = jnp.concatenate([p10, p10, p10, p10], axis=0)
    xb = x_ref[...].reshape(R // 40, 40, 128)
    o_ref[...] = (xb + pos40[None, :, :]).reshape(R, 128)


def kernel(x, pos_emb_table):
    B, S, D = x.shape
    N = B * S
    x2 = x.reshape(N, D)
    R = 5120
    grid = (N // R,)
    out = pl.pallas_call(
        _body,
        grid=grid,
        in_specs=[
            pl.BlockSpec((R, D), lambda i: (i, 0)),
            pl.BlockSpec((16, D), lambda i: (0, 0)),
        ],
        out_specs=pl.BlockSpec((R, D), lambda i: (i, 0)),
        out_shape=jax.ShapeDtypeStruct((N, D), x.dtype),
        compiler_params=pltpu.CompilerParams(
            dimension_semantics=("arbitrary",),
        ),
    )(x2, pos_emb_table)
    return out.reshape(B, S, D)


# speedup vs baseline: 1.7502x; 1.7502x over previous
import jax
import jax.numpy as jnp
from jax.experimental import pallas as pl
from jax.experimental.pallas import tpu as pltpu


def _body(x_ref, pos_ref, o_ref):
    o_ref[...] = x_ref[...] + pos_ref[0:10, :]


def kernel(x, pos_emb_table):
    B, S, D = x.shape
    BB = 512
    grid = (B // BB,)
    return pl.pallas_call(
        _body,
        grid=grid,
        in_specs=[
            pl.BlockSpec((BB, S, D), lambda i: (i, 0, 0)),
            pl.BlockSpec((16, D), lambda i: (0, 0)),
        ],
        out_specs=pl.BlockSpec((BB, S, D), lambda i: (i, 0, 0)),
        out_shape=jax.ShapeDtypeStruct((B, S, D), x.dtype),
        compiler_params=pltpu.CompilerParams(
            dimension_semantics=("parallel",),
        ),
    )(x, pos_emb_table)


# speedup vs baseline: 2.0282x; 1.1588x over previous
import jax
import jax.numpy as jnp
from jax.experimental import pallas as pl
from jax.experimental.pallas import tpu as pltpu

_CH = 256
_K = 8


def _body(x_hbm, pos_ref, o_hbm, bufs, obufs, in_sems, out_sems):
    i = pl.program_id(0)
    n = pl.num_programs(0)
    s = jax.lax.rem(i, _K)

    @pl.when(i == 0)
    def _prologue():
        for k in range(_K):
            pltpu.make_async_copy(
                x_hbm.at[pl.ds(k * _CH, _CH)], bufs.at[k], in_sems.at[k]
            ).start()

    @pl.when(i >= _K)
    def _wait_out():
        pltpu.make_async_copy(
            obufs.at[s], o_hbm.at[pl.ds(0, _CH)], out_sems.at[s]
        ).wait()

    pltpu.make_async_copy(
        x_hbm.at[pl.ds(i * _CH, _CH)], bufs.at[s], in_sems.at[s]
    ).wait()

    obufs[s] = bufs[s] + pos_ref[0:10, :]

    pltpu.make_async_copy(
        obufs.at[s], o_hbm.at[pl.ds(i * _CH, _CH)], out_sems.at[s]
    ).start()

    @pl.when(i + _K < n)
    def _prefetch():
        pltpu.make_async_copy(
            x_hbm.at[pl.ds((i + _K) * _CH, _CH)], bufs.at[s], in_sems.at[s]
        ).start()

    @pl.when(i == n - 1)
    def _epilogue():
        for k in range(_K):
            pltpu.make_async_copy(
                obufs.at[k], o_hbm.at[pl.ds(0, _CH)], out_sems.at[k]
            ).wait()


def kernel(x, pos_emb_table):
    B, S, D = x.shape
    grid = (B // _CH,)
    return pl.pallas_call(
        _body,
        grid=grid,
        in_specs=[
            pl.BlockSpec(memory_space=pl.ANY),
            pl.BlockSpec((16, D), lambda i: (0, 0)),
        ],
        out_specs=pl.BlockSpec(memory_space=pl.ANY),
        out_shape=jax.ShapeDtypeStruct((B, S, D), x.dtype),
        scratch_shapes=[
            pltpu.VMEM((_K, _CH, S, D), x.dtype),
            pltpu.VMEM((_K, _CH, S, D), x.dtype),
            pltpu.SemaphoreType.DMA((_K,)),
            pltpu.SemaphoreType.DMA((_K,)),
        ],
        compiler_params=pltpu.CompilerParams(
            dimension_semantics=("arbitrary",),
        ),
    )(x, pos_emb_table)
